# baseline (speedup 1.0000x reference)
import functools

import jax
import jax.numpy as jnp
from jax import lax
from jax.experimental import pallas as pl
from jax.experimental.pallas import tpu as pltpu
from jax.experimental.pallas import tpu_sc as plsc

_N = 10000
_E = 320000
_D = 128
_DIM = 8

_NC = 2
_NS = 16
_NW = _NC * _NS
_EPT = _E // _NW
_C = 80
_NCHUNK = _EPT // _C
_NP = 10240
_RPT = _NP // _NS


def _agg_body(x_hbm, src_hbm, dst_hbm, g_hbm, out_hbm,
              src_v, dst_b, g_b, rows_a, rows_b, acc,
              sem_a, sem_b, sem_p0, sem_p1):
    c = lax.axis_index("c")
    s = lax.axis_index("s")
    wid = c * _NS + s

    def _zrow(r, _):
        for j in range(_D // 16):
            rows_a[r, pl.ds(j * 16, 16)] = jnp.zeros((16,), jnp.float32)
        return _
    lax.fori_loop(0, _C, _zrow, None)
    row0 = s * _RPT
    for kz in range(_RPT // _C):
        pltpu.sync_copy(rows_a, acc.at[pl.ds(row0 + kz * _C, _C)])

    pltpu.sync_copy(src_hbm.at[pl.ds(wid * _EPT, _EPT)], src_v)
    plsc.subcore_barrier()

    def _gather(k, buf, sem):
        pltpu.async_copy(x_hbm.at[src_v.at[pl.ds(k * _C, _C)]], buf, sem)

    def _gwait(buf, sem):
        pltpu.make_async_copy(x_hbm.at[src_v.at[pl.ds(0, _C)]], buf, sem).wait()

    def _prefetch(k, p, sem):
        off = wid * _EPT + k * _C
        pltpu.async_copy(dst_hbm.at[pl.ds(off, _C)], dst_b.at[p], sem)
        pltpu.async_copy(g_hbm.at[pl.ds(off, _C)], g_b.at[p], sem)

    def _pwait(p, sem):
        pltpu.make_async_copy(dst_hbm.at[pl.ds(0, _C)], dst_b.at[p], sem).wait()
        pltpu.make_async_copy(g_hbm.at[pl.ds(0, _C)], g_b.at[p], sem).wait()

    def _scale_and_scatter(p, buf):
        for gi in range(_C // 16):
            gvec = g_b[p, pl.ds(gi * 16, 16)]
            for li in range(16):
                gv = gvec.at[jnp.full((16,), li, jnp.int32)].get(
                    mode='promise_in_bounds')
                e = gi * 16 + li
                for j in range(_D // 16):
                    sl = pl.ds(j * 16, 16)
                    buf[e, sl] = buf[e, sl] * gv
        pltpu.sync_copy(buf, acc.at[dst_b.at[p]], add=True)

    _prefetch(0, 0, sem_p0)
    _gather(0, rows_a, sem_a)

    def _pair(i, _):
        ka = 2 * i
        _prefetch(ka + 1, 1, sem_p1)
        _gather(ka + 1, rows_b, sem_b)
        _pwait(0, sem_p0)
        _gwait(rows_a, sem_a)
        _scale_and_scatter(0, rows_a)
        _prefetch(ka + 2, 0, sem_p0)
        _gather(ka + 2, rows_a, sem_a)
        _pwait(1, sem_p1)
        _gwait(rows_b, sem_b)
        _scale_and_scatter(1, rows_b)
        return _
    lax.fori_loop(0, (_NCHUNK - 1) // 2, _pair, None)

    _pwait(0, sem_p0)
    _gwait(rows_a, sem_a)
    _scale_and_scatter(0, rows_a)

    plsc.subcore_barrier()
    obase = c * _NP + s * _RPT
    pltpu.sync_copy(acc.at[pl.ds(s * _RPT, _RPT)], out_hbm.at[pl.ds(obase, _RPT)])


def _make_agg():
    mesh = plsc.VectorSubcoreMesh(core_axis_name="c", subcore_axis_name="s",
                                  num_cores=_NC, num_subcores=_NS)
    return pl.kernel(
        _agg_body,
        out_type=jax.ShapeDtypeStruct((_NC * _NP, _D), jnp.float32),
        mesh=mesh,
        scratch_types=[
            pltpu.VMEM((_EPT,), jnp.int32),
            pltpu.VMEM((2, _C), jnp.int32),
            pltpu.VMEM((2, _C), jnp.float32),
            pltpu.VMEM((_C, _D), jnp.float32),
            pltpu.VMEM((_C, _D), jnp.float32),
            pltpu.VMEM_SHARED((_NP, _D), jnp.float32),
            pltpu.SemaphoreType.DMA,
            pltpu.SemaphoreType.DMA,
            pltpu.SemaphoreType.DMA,
            pltpu.SemaphoreType.DMA,
        ],
    )


_agg = _make_agg()


def _gauss_tc(pwt_ref, mu1_ref, is1_ref, mu2_ref, is2_ref, out_ref):
    pwt = pwt_ref[...]
    d1 = pwt - mu1_ref[...]
    e1 = jnp.sum(d1 * d1 * (is1_ref[...] ** 2), axis=0, keepdims=True)
    d2 = pwt - mu2_ref[...]
    e2 = jnp.sum(d2 * d2 * (is2_ref[...] ** 2), axis=0, keepdims=True)
    out_ref[...] = jnp.exp(-0.5 * jnp.concatenate([e1, e2], axis=0))


def _gaussians(pseudo_t, mu1, is1, mu2, is2):
    B = 32000
    grid = _E // B
    small = pl.BlockSpec((_DIM, 1), lambda i: (0, 0))
    return pl.pallas_call(
        _gauss_tc,
        grid=(grid,),
        in_specs=[pl.BlockSpec((_DIM, B), lambda i: (0, i)),
                  small, small, small, small],
        out_specs=pl.BlockSpec((2, B), lambda i: (0, i)),
        out_shape=jax.ShapeDtypeStruct((2, _E), jnp.float32),
    )(pseudo_t, mu1.reshape(_DIM, 1), is1.reshape(_DIM, 1),
      mu2.reshape(_DIM, 1), is2.reshape(_DIM, 1))


def _mm_tc(p_ref, w_ref, b_ref, out_ref, *, softmax):
    x = p_ref[0] + p_ref[1]
    h = jnp.dot(x, w_ref[...], preferred_element_type=jnp.float32) + b_ref[...]
    if softmax:
        m = jnp.max(h, axis=1, keepdims=True)
        lse = jnp.log(jnp.sum(jnp.exp(h - m), axis=1, keepdims=True)) + m
        h = h - lse
    out_ref[...] = h


def _matmul(p2, W, b, softmax):
    Bn = 2000
    grid = _N // Bn
    return pl.pallas_call(
        functools.partial(_mm_tc, softmax=softmax),
        grid=(grid,),
        in_specs=[pl.BlockSpec((2, Bn, _D), lambda i: (0, i, 0)),
                  pl.BlockSpec((_D, _D), lambda i: (0, 0)),
                  pl.BlockSpec((1, _D), lambda i: (0, 0))],
        out_specs=pl.BlockSpec((Bn, _D), lambda i: (i, 0)),
        out_shape=jax.ShapeDtypeStruct((_N, _D), jnp.float32),
    )(p2, W, b.reshape(1, _D))


def kernel(edge_index, edge_weight, features, W1, mu1, inv_sigma1, b1,
           W2, mu2, inv_sigma2, b2):
    src = edge_index[0]
    dst = edge_index[1]
    g12 = _gaussians(edge_weight.T, mu1, inv_sigma1, mu2, inv_sigma2)
    g1 = g12[0]
    g2 = g12[1]

    p1 = _agg(features, src, dst, g1).reshape(_NC, _NP, _D)
    x1 = _matmul(p1, W1, b1, softmax=False)
    p2 = _agg(x1, src, dst, g2).reshape(_NC, _NP, _D)
    return _matmul(p2, W2, b2, softmax=True)


# speedup vs baseline: 1.1850x; 1.1850x over previous
import functools

import jax
import jax.numpy as jnp
from jax import lax
from jax.experimental import pallas as pl
from jax.experimental.pallas import tpu as pltpu
from jax.experimental.pallas import tpu_sc as plsc

_N = 10000
_E = 320000
_D = 128
_DIM = 8

_NC = 2
_NS = 16
_NW = _NC * _NS
_EPT = _E // _NW
_C = 80
_NCHUNK = _EPT // _C
_NP = 10240
_RPT = _NP // _NS


def _agg_body(x_hbm, src_hbm, dst_hbm, g_hbm, out_hbm,
              src_v, dst_b, g_b, rows_a, rows_b, rows_c, acc,
              gsem_a, gsem_b, gsem_c, psem_a, psem_b, psem_c,
              ssem_a, ssem_b, ssem_c):
    c = lax.axis_index("c")
    s = lax.axis_index("s")
    wid = c * _NS + s

    def _zrow(r, _):
        for j in range(_D // 16):
            rows_a[r, pl.ds(j * 16, 16)] = jnp.zeros((16,), jnp.float32)
        return _
    lax.fori_loop(0, _C, _zrow, None)
    row0 = s * _RPT
    for kz in range(_RPT // _C):
        pltpu.sync_copy(rows_a, acc.at[pl.ds(row0 + kz * _C, _C)])

    pltpu.sync_copy(src_hbm.at[pl.ds(wid * _EPT, _EPT)], src_v)
    plsc.subcore_barrier()

    def _gather(k, buf, sem):
        pltpu.async_copy(x_hbm.at[src_v.at[pl.ds(k * _C, _C)]], buf, sem)

    def _gwait(buf, sem):
        pltpu.make_async_copy(x_hbm.at[src_v.at[pl.ds(0, _C)]], buf, sem).wait()

    def _prefetch(k, p, sem):
        off = wid * _EPT + k * _C
        pltpu.async_copy(dst_hbm.at[pl.ds(off, _C)], dst_b.at[p], sem)
        pltpu.async_copy(g_hbm.at[pl.ds(off, _C)], g_b.at[p], sem)

    def _pwait(p, sem):
        pltpu.make_async_copy(dst_hbm.at[pl.ds(0, _C)], dst_b.at[p], sem).wait()
        pltpu.make_async_copy(g_hbm.at[pl.ds(0, _C)], g_b.at[p], sem).wait()

    def _scale(p, buf):
        for gi in range(_C // 16):
            gvec = g_b[p, pl.ds(gi * 16, 16)]

            def _lane(t, gvec=gvec, buf=buf, gi=gi):
                for u in range(4):
                    li = 4 * t + u
                    gv = gvec.at[jnp.full((16,), li, jnp.int32)].get(
                        mode='promise_in_bounds')
                    e = gi * 16 + li
                    for j in range(_D // 16):
                        sl = pl.ds(j * 16, 16)
                        buf[e, sl] = buf[e, sl] * gv
                return t

            lax.fori_loop(0, 4, lambda t, _, f=_lane: (f(t), None)[1], None)

    def _scatter(p, buf, sem):
        pltpu.async_copy(buf, acc.at[dst_b.at[p]], sem)

    def _swait(p, buf, sem):
        pltpu.make_async_copy(buf, acc.at[dst_b.at[p]], sem).wait()

    bufs = (rows_a, rows_b, rows_c)
    gsems = (gsem_a, gsem_b, gsem_c)
    psems = (psem_a, psem_b, psem_c)
    ssems = (ssem_a, ssem_b, ssem_c)

    def _pg(k, x):
        _prefetch(k, x, psems[x])
        _gather(k, bufs[x], gsems[x])

    def _slot(k, x, swait_prev=True, regather=True):
        y = (x + 2) % 3
        _pwait(x, psems[x])
        _gwait(bufs[x], gsems[x])
        _scale(x, bufs[x])
        _scatter(x, bufs[x], ssems[x])
        if swait_prev:
            _swait(y, bufs[y], ssems[y])
        if regather:
            _pg(k + 2, y)

    _pg(0, 0)
    _pg(1, 1)
    _slot(0, 0, swait_prev=False)
    _slot(1, 1)
    _slot(2, 2)

    def _trip(i, _):
        k = 3 * i
        _slot(k, 0)
        _slot(k + 1, 1)
        _slot(k + 2, 2)
        return _
    lax.fori_loop(1, (_NCHUNK - 2) // 3, _trip, None)

    _slot(_NCHUNK - 2, 0, regather=False)
    _slot(_NCHUNK - 1, 1, regather=False)
    _swait(1, rows_b, ssem_b)

    plsc.subcore_barrier()
    obase = c * _NP + s * _RPT
    pltpu.sync_copy(acc.at[pl.ds(s * _RPT, _RPT)], out_hbm.at[pl.ds(obase, _RPT)])


def _make_agg():
    mesh = plsc.VectorSubcoreMesh(core_axis_name="c", subcore_axis_name="s",
                                  num_cores=_NC, num_subcores=_NS)
    return pl.kernel(
        _agg_body,
        out_type=jax.ShapeDtypeStruct((_NC * _NP, _D), jnp.float32),
        mesh=mesh,
        scratch_types=[
            pltpu.VMEM((_EPT,), jnp.int32),
            pltpu.VMEM((3, _C), jnp.int32),
            pltpu.VMEM((3, _C), jnp.float32),
            pltpu.VMEM((_C, _D), jnp.float32),
            pltpu.VMEM((_C, _D), jnp.float32),
            pltpu.VMEM((_C, _D), jnp.float32),
            pltpu.VMEM_SHARED((_NP, _D), jnp.float32),
        ] + [pltpu.SemaphoreType.DMA] * 9,
    )


_agg = _make_agg()


def _gauss_tc(pwt_ref, mu1_ref, is1_ref, mu2_ref, is2_ref, out_ref):
    pwt = pwt_ref[...]
    d1 = pwt - mu1_ref[...]
    e1 = jnp.sum(d1 * d1 * (is1_ref[...] ** 2), axis=0, keepdims=True)
    d2 = pwt - mu2_ref[...]
    e2 = jnp.sum(d2 * d2 * (is2_ref[...] ** 2), axis=0, keepdims=True)
    out_ref[...] = jnp.exp(-0.5 * jnp.concatenate([e1, e2], axis=0))


def _gaussians(pseudo_t, mu1, is1, mu2, is2):
    B = 32000
    grid = _E // B
    small = pl.BlockSpec((_DIM, 1), lambda i: (0, 0))
    return pl.pallas_call(
        _gauss_tc,
        grid=(grid,),
        in_specs=[pl.BlockSpec((_DIM, B), lambda i: (0, i)),
                  small, small, small, small],
        out_specs=pl.BlockSpec((2, B), lambda i: (0, i)),
        out_shape=jax.ShapeDtypeStruct((2, _E), jnp.float32),
    )(pseudo_t, mu1.reshape(_DIM, 1), is1.reshape(_DIM, 1),
      mu2.reshape(_DIM, 1), is2.reshape(_DIM, 1))


def _mm_tc(p_ref, w_ref, b_ref, out_ref, *, softmax):
    x = p_ref[0] + p_ref[1]
    h = jnp.dot(x, w_ref[...], preferred_element_type=jnp.float32) + b_ref[...]
    if softmax:
        m = jnp.max(h, axis=1, keepdims=True)
        lse = jnp.log(jnp.sum(jnp.exp(h - m), axis=1, keepdims=True)) + m
        h = h - lse
    out_ref[...] = h


def _matmul(p2, W, b, softmax):
    Bn = 2000
    grid = _N // Bn
    return pl.pallas_call(
        functools.partial(_mm_tc, softmax=softmax),
        grid=(grid,),
        in_specs=[pl.BlockSpec((2, Bn, _D), lambda i: (0, i, 0)),
                  pl.BlockSpec((_D, _D), lambda i: (0, 0)),
                  pl.BlockSpec((1, _D), lambda i: (0, 0))],
        out_specs=pl.BlockSpec((Bn, _D), lambda i: (i, 0)),
        out_shape=jax.ShapeDtypeStruct((_N, _D), jnp.float32),
    )(p2, W, b.reshape(1, _D))


def kernel(edge_index, edge_weight, features, W1, mu1, inv_sigma1, b1,
           W2, mu2, inv_sigma2, b2):
    src = edge_index[0]
    dst = edge_index[1]
    g12 = _gaussians(edge_weight.T, mu1, inv_sigma1, mu2, inv_sigma2)
    g1 = g12[0]
    g2 = g12[1]

    p1 = _agg(features, src, dst, g1).reshape(_NC, _NP, _D)
    x1 = _matmul(p1, W1, b1, softmax=False)
    p2 = _agg(x1, src, dst, g2).reshape(_NC, _NP, _D)
    return _matmul(p2, W2, b2, softmax=True)
